# baseline (speedup 1.0000x reference)
import jax
import jax.numpy as jnp
from jax.experimental import pallas as pl
from jax.experimental.pallas import tpu as pltpu

LN_EPS = 1e-12
_NSLOT = 6


def _emb_kernel(tt_ref, mk_ref, pos_ref, rows_ref, inp_hbm, img_hbm,
                out_hbm, mask_ref, ibuf, gbuf, slab, isem, gsem, osem):
    B, L, H = inp_hbm.shape
    NIMG = img_hbm.shape[1]

    def in_copy(g, s):
        return pltpu.make_async_copy(inp_hbm.at[g], ibuf.at[s], isem.at[s])

    def im_copy(g, s):
        return pltpu.make_async_copy(img_hbm.at[g], gbuf.at[s], gsem.at[s])

    def out_copy(g, s):
        return pltpu.make_async_copy(slab.at[s], out_hbm.at[g], osem.at[s])

    for g in range(_NSLOT):
        in_copy(g, g).start()
        im_copy(g, g).start()

    mask_ref[...] = jnp.concatenate(
        [jnp.ones((B, 1, 1), jnp.int32), mk_ref[...]], axis=2)

    row0 = rows_ref[0:1, :]
    diff = rows_ref[1:2, :]
    gam = rows_ref[2:3, :]
    b2 = rows_ref[3:4, :]
    pos = pos_ref[...]

    def step(g, _):
        s = jax.lax.rem(g, _NSLOT)
        in_copy(g, s).wait()

        @pl.when(g >= _NSLOT)
        def _():
            out_copy(g - _NSLOT, s).wait()

        slab[s, 0:L, :] = ibuf[s]
        im_copy(g, s).wait()
        slab[s, L:1 + L + 575, :] = gbuf[s]
        out_copy(g, s).start()

        @pl.when(g + _NSLOT < B)
        def _():
            in_copy(g + _NSLOT, s).start()
            im_copy(g + _NSLOT, s).start()

        return 0

    jax.lax.fori_loop(0, B, step, 0)

    for g in range(B - _NSLOT, B):
        out_copy(g, g % _NSLOT).wait()


def kernel(input_ids, attention_mask, token_type_ids, pixel_values, pixel_mask,
           inputs_embeds, image_embeds, image_token_type_idx,
           text_pos_emb, text_tok_type_emb, ln_gamma, ln_beta,
           cls_token, modality_tok_type_emb):
    B, L, H = inputs_embeds.shape
    NIMG = image_embeds.shape[1]
    S = 1 + L + NIMG

    mi = jnp.take(modality_tok_type_emb, image_token_type_idx, axis=0).reshape(1, H)
    b2 = (ln_beta + modality_tok_type_emb[0]).reshape(1, H)
    row0 = text_tok_type_emb[0:1, :]
    diff = text_tok_type_emb[1:2, :] - row0
    rows8 = jnp.concatenate(
        [row0, diff, ln_gamma.reshape(1, H), b2,
         cls_token.reshape(1, H), mi, jnp.zeros((2, H), jnp.float32)], axis=0)
    ttf3 = token_type_ids.astype(jnp.float32).reshape(B, L, 1)
    mk3 = jnp.concatenate([attention_mask, pixel_mask], axis=1).reshape(B, 1, L + NIMG)

    out, mask3 = pl.pallas_call(
        _emb_kernel,
        grid=(1,),
        in_specs=[
            pl.BlockSpec((B, L, 1), lambda b: (0, 0, 0)),
            pl.BlockSpec((B, 1, L + NIMG), lambda b: (0, 0, 0)),
            pl.BlockSpec((L, H), lambda b: (0, 0)),
            pl.BlockSpec((8, H), lambda b: (0, 0)),
            pl.BlockSpec(memory_space=pl.MemorySpace.ANY),
            pl.BlockSpec(memory_space=pl.MemorySpace.ANY),
        ],
        out_specs=[
            pl.BlockSpec(memory_space=pl.MemorySpace.ANY),
            pl.BlockSpec((B, 1, S), lambda b: (0, 0, 0)),
        ],
        out_shape=[
            jax.ShapeDtypeStruct((B, S, H), jnp.float32),
            jax.ShapeDtypeStruct((B, 1, S), jnp.int32),
        ],
        scratch_shapes=[
            pltpu.VMEM((_NSLOT, L, H), jnp.float32),
            pltpu.VMEM((_NSLOT, NIMG, H), jnp.float32),
            pltpu.VMEM((_NSLOT, S, H), jnp.float32),
            pltpu.SemaphoreType.DMA((_NSLOT,)),
            pltpu.SemaphoreType.DMA((_NSLOT,)),
            pltpu.SemaphoreType.DMA((_NSLOT,)),
        ],
    )(ttf3, mk3, text_pos_emb[:L], rows8, inputs_embeds, image_embeds)

    return out, mask3.reshape(B, S)


# speedup vs baseline: 1.0039x; 1.0039x over previous
import jax
import jax.numpy as jnp
from jax.experimental import pallas as pl
from jax.experimental.pallas import tpu as pltpu

LN_EPS = 1e-12
_NSLOT = 4


def _emb_kernel(tt_ref, mk_ref, pos_ref, rows_ref, inp_hbm, img_hbm,
                out_hbm, mask_ref, ibuf, gbuf, slab, isem, gsem, osem):
    B, L, H = inp_hbm.shape
    NIMG = img_hbm.shape[1]

    def in_copy(g, s):
        return pltpu.make_async_copy(inp_hbm.at[g], ibuf.at[s], isem.at[s])

    def im_copy(g, s):
        return pltpu.make_async_copy(img_hbm.at[g], gbuf.at[s], gsem.at[s])

    def out_copy(g, s):
        return pltpu.make_async_copy(slab.at[s], out_hbm.at[g], osem.at[s])

    for g in range(_NSLOT):
        in_copy(g, g).start()
        im_copy(g, g).start()

    mask_ref[...] = jnp.concatenate(
        [jnp.ones((B, 1, 1), jnp.int32), mk_ref[...]], axis=2)

    row0 = rows_ref[0:1, :]
    diff = rows_ref[1:2, :]
    gam = rows_ref[2:3, :]
    b2 = rows_ref[3:4, :]
    pos = pos_ref[...]

    def step(g, _):
        s = jax.lax.rem(g, _NSLOT)
        in_copy(g, s).wait()

        @pl.when(g >= _NSLOT)
        def _():
            out_copy(g - _NSLOT, s).wait()

        ttf = tt_ref[g]
        emb = ibuf[s] + pos + (row0 + ttf * diff)
        mu = jnp.mean(emb, axis=1, keepdims=True)
        d = emb - mu
        var = jnp.mean(d * d, axis=1, keepdims=True)
        slab[s, 0:1, :] = rows_ref[4:5, :]
        slab[s, 1:1 + L, :] = gam * d * jax.lax.rsqrt(var + LN_EPS) + b2
        im_copy(g, s).wait()
        slab[s, 1 + L:, :] = gbuf[s] + rows_ref[5:6, :]
        out_copy(g, s).start()

        @pl.when(g + _NSLOT < B)
        def _():
            in_copy(g + _NSLOT, s).start()
            im_copy(g + _NSLOT, s).start()

        return 0

    jax.lax.fori_loop(0, B, step, 0)

    for g in range(B - _NSLOT, B):
        out_copy(g, g % _NSLOT).wait()


def kernel(input_ids, attention_mask, token_type_ids, pixel_values, pixel_mask,
           inputs_embeds, image_embeds, image_token_type_idx,
           text_pos_emb, text_tok_type_emb, ln_gamma, ln_beta,
           cls_token, modality_tok_type_emb):
    B, L, H = inputs_embeds.shape
    NIMG = image_embeds.shape[1]
    S = 1 + L + NIMG

    mi = jnp.take(modality_tok_type_emb, image_token_type_idx, axis=0).reshape(1, H)
    b2 = (ln_beta + modality_tok_type_emb[0]).reshape(1, H)
    row0 = text_tok_type_emb[0:1, :]
    diff = text_tok_type_emb[1:2, :] - row0
    rows8 = jnp.concatenate(
        [row0, diff, ln_gamma.reshape(1, H), b2,
         cls_token.reshape(1, H), mi, jnp.zeros((2, H), jnp.float32)], axis=0)
    ttf3 = token_type_ids.astype(jnp.float32).reshape(B, L, 1)
    mk3 = jnp.concatenate([attention_mask, pixel_mask], axis=1).reshape(B, 1, L + NIMG)

    out, mask3 = pl.pallas_call(
        _emb_kernel,
        grid=(1,),
        in_specs=[
            pl.BlockSpec((B, L, 1), lambda b: (0, 0, 0)),
            pl.BlockSpec((B, 1, L + NIMG), lambda b: (0, 0, 0)),
            pl.BlockSpec((L, H), lambda b: (0, 0)),
            pl.BlockSpec((8, H), lambda b: (0, 0)),
            pl.BlockSpec(memory_space=pl.MemorySpace.ANY),
            pl.BlockSpec(memory_space=pl.MemorySpace.ANY),
        ],
        out_specs=[
            pl.BlockSpec(memory_space=pl.MemorySpace.ANY),
            pl.BlockSpec((B, 1, S), lambda b: (0, 0, 0)),
        ],
        out_shape=[
            jax.ShapeDtypeStruct((B, S, H), jnp.float32),
            jax.ShapeDtypeStruct((B, 1, S), jnp.int32),
        ],
        scratch_shapes=[
            pltpu.VMEM((_NSLOT, L, H), jnp.float32),
            pltpu.VMEM((_NSLOT, NIMG, H), jnp.float32),
            pltpu.VMEM((_NSLOT, S, H), jnp.float32),
            pltpu.SemaphoreType.DMA((_NSLOT,)),
            pltpu.SemaphoreType.DMA((_NSLOT,)),
            pltpu.SemaphoreType.DMA((_NSLOT,)),
        ],
    )(ttf3, mk3, text_pos_emb[:L], rows8, inputs_embeds, image_embeds)

    return out, mask3.reshape(B, S)


# speedup vs baseline: 1.0041x; 1.0002x over previous
import jax
import jax.numpy as jnp
from jax.experimental import pallas as pl
from jax.experimental.pallas import tpu as pltpu

LN_EPS = 1e-12
_NSLOT = 4


def _emb_kernel(tt_ref, mk_ref, pos_ref, rows_ref, inp_hbm, img_hbm,
                out_hbm, mask_ref, ibuf, gbuf, slab, isem, gsem, osem):
    B, L, H = inp_hbm.shape
    NIMG = img_hbm.shape[1]

    def in_copy(g, s):
        return pltpu.make_async_copy(inp_hbm.at[g], ibuf.at[s], isem.at[s])

    def im_copy(g, s):
        return pltpu.make_async_copy(img_hbm.at[g], gbuf.at[s], gsem.at[s])

    def out_copy(g, s):
        return pltpu.make_async_copy(slab.at[s], out_hbm.at[g], osem.at[s])

    for g in range(_NSLOT):
        in_copy(g, g).start()
        im_copy(g, g).start()

    mask_ref[...] = jnp.concatenate(
        [jnp.ones((B, 1, 1), jnp.int32), mk_ref[...]], axis=2)

    row0 = rows_ref[0:1, :]
    diff = rows_ref[1:2, :]
    gam = rows_ref[2:3, :]
    b2 = rows_ref[3:4, :]
    pos = pos_ref[...]

    def step(g, s):
        in_copy(g, s).wait()

        if g >= _NSLOT:
            out_copy(g - _NSLOT, s).wait()

        ttf = tt_ref[g]
        emb = ibuf[s] + pos + (row0 + ttf * diff)
        mu = jnp.mean(emb, axis=1, keepdims=True)
        d = emb - mu
        var = jnp.mean(d * d, axis=1, keepdims=True)
        slab[s, 0:1, :] = rows_ref[4:5, :]
        slab[s, 1:1 + L, :] = gam * d * jax.lax.rsqrt(var + LN_EPS) + b2
        im_copy(g, s).wait()
        slab[s, 1 + L:, :] = gbuf[s] + rows_ref[5:6, :]
        out_copy(g, s).start()

        if g + _NSLOT < B:
            in_copy(g + _NSLOT, s).start()
            im_copy(g + _NSLOT, s).start()

    for g in range(B):
        step(g, g % _NSLOT)

    for g in range(B - _NSLOT, B):
        out_copy(g, g % _NSLOT).wait()


def kernel(input_ids, attention_mask, token_type_ids, pixel_values, pixel_mask,
           inputs_embeds, image_embeds, image_token_type_idx,
           text_pos_emb, text_tok_type_emb, ln_gamma, ln_beta,
           cls_token, modality_tok_type_emb):
    B, L, H = inputs_embeds.shape
    NIMG = image_embeds.shape[1]
    S = 1 + L + NIMG

    mi = jnp.take(modality_tok_type_emb, image_token_type_idx, axis=0).reshape(1, H)
    b2 = (ln_beta + modality_tok_type_emb[0]).reshape(1, H)
    row0 = text_tok_type_emb[0:1, :]
    diff = text_tok_type_emb[1:2, :] - row0
    rows8 = jnp.concatenate(
        [row0, diff, ln_gamma.reshape(1, H), b2,
         cls_token.reshape(1, H), mi, jnp.zeros((2, H), jnp.float32)], axis=0)
    ttf3 = token_type_ids.astype(jnp.float32).reshape(B, L, 1)
    mk3 = jnp.concatenate([attention_mask, pixel_mask], axis=1).reshape(B, 1, L + NIMG)

    out, mask3 = pl.pallas_call(
        _emb_kernel,
        grid=(1,),
        in_specs=[
            pl.BlockSpec((B, L, 1), lambda b: (0, 0, 0)),
            pl.BlockSpec((B, 1, L + NIMG), lambda b: (0, 0, 0)),
            pl.BlockSpec((L, H), lambda b: (0, 0)),
            pl.BlockSpec((8, H), lambda b: (0, 0)),
            pl.BlockSpec(memory_space=pl.MemorySpace.ANY),
            pl.BlockSpec(memory_space=pl.MemorySpace.ANY),
        ],
        out_specs=[
            pl.BlockSpec(memory_space=pl.MemorySpace.ANY),
            pl.BlockSpec((B, 1, S), lambda b: (0, 0, 0)),
        ],
        out_shape=[
            jax.ShapeDtypeStruct((B, S, H), jnp.float32),
            jax.ShapeDtypeStruct((B, 1, S), jnp.int32),
        ],
        scratch_shapes=[
            pltpu.VMEM((_NSLOT, L, H), jnp.float32),
            pltpu.VMEM((_NSLOT, NIMG, H), jnp.float32),
            pltpu.VMEM((_NSLOT, S, H), jnp.float32),
            pltpu.SemaphoreType.DMA((_NSLOT,)),
            pltpu.SemaphoreType.DMA((_NSLOT,)),
            pltpu.SemaphoreType.DMA((_NSLOT,)),
        ],
    )(ttf3, mk3, text_pos_emb[:L], rows8, inputs_embeds, image_embeds)

    return out, mask3.reshape(B, S)
